# baseline (speedup 1.0000x reference)
import functools

import jax
import jax.numpy as jnp
from jax import lax
from jax.experimental import pallas as pl
from jax.experimental.pallas import tpu as pltpu
from jax.experimental.pallas import tpu_sc as plsc

_EMB = 50
_B = 16384
_NW = 32
_BPW = _B // _NW
_CH = 128
_NCH = _BPW // _CH
_L = 16


def _sc_gather_body(uidx_hbm, midx_hbm, utab_hbm, mtab_hbm,
                    wu_out, wm_out, su_out, sm_out,
                    uidx_st, midx_st, su_v, sm_v, gidx_v, slab_v, sem):
    wid = lax.axis_index("s") * 2 + lax.axis_index("c")
    base = wid * _BPW
    pltpu.sync_copy(uidx_hbm.at[pl.ds(base, _BPW)], uidx_st)
    pltpu.sync_copy(midx_hbm.at[pl.ds(base, _BPW)], midx_st)

    for t, (idx_st, s_v) in enumerate(((uidx_st, su_v), (midx_st, sm_v))):
        for g in range(_BPW // _L):
            v = idx_st[pl.ds(_L * g, _L)]
            tt = v * 25
            g0 = lax.shift_right_logical(tt, 3)
            s_v[pl.ds(_L * g, _L)] = lax.bitwise_and(tt, 7)
            ch, lo = divmod(g, _CH // _L)
            for j in range(4):
                gidx_v[t, j, ch, pl.ds(_L * lo, _L)] = g0 + j

    cps = []
    for t, tab in enumerate((utab_hbm, mtab_hbm)):
        for ch in range(_NCH):
            for j in range(4):
                cps.append(pltpu.async_copy(
                    tab.at[gidx_v.at[t, j, ch]],
                    slab_v.at[t, j, pl.ds(_CH * ch, _CH)], sem))
    for cp in cps:
        cp.wait()

    for t, w_out in enumerate((wu_out, wm_out)):
        for j in range(4):
            pltpu.sync_copy(
                slab_v.at[t, j],
                w_out.at[pl.ds(base, _BPW), pl.ds(_L * j, _L)])
    pltpu.sync_copy(su_v, su_out.at[pl.ds(base, _BPW)])
    pltpu.sync_copy(sm_v, sm_out.at[pl.ds(base, _BPW)])


@functools.cache
def _sc_gather():
    return pl.kernel(
        _sc_gather_body,
        out_type=(jax.ShapeDtypeStruct((_B, 4 * _L), jnp.float32),
                  jax.ShapeDtypeStruct((_B, 4 * _L), jnp.float32),
                  jax.ShapeDtypeStruct((_B,), jnp.int32),
                  jax.ShapeDtypeStruct((_B,), jnp.int32)),
        mesh=plsc.VectorSubcoreMesh(core_axis_name="c", subcore_axis_name="s"),
        scratch_types=[
            pltpu.VMEM((_BPW,), jnp.int32),
            pltpu.VMEM((_BPW,), jnp.int32),
            pltpu.VMEM((_BPW,), jnp.int32),
            pltpu.VMEM((_BPW,), jnp.int32),
            pltpu.VMEM((2, 4, _NCH, _CH), jnp.int32),
            pltpu.VMEM((2, 4, _BPW, _L), jnp.float32),
            pltpu.SemaphoreType.DMA,
        ],
        compiler_params=pltpu.CompilerParams(use_tc_tiling_on_sc=False),
    )


def _mlp_body(wu_ref, wm_ref, g_ref, su_ref, sm_ref, wua_ref, wma_ref,
              w1g_ref, b1_ref, w2t_ref, b2_ref, o_ref):
    pu = jnp.dot(wu_ref[...], wua_ref[...], preferred_element_type=jnp.float32)
    pm = jnp.dot(wm_ref[...], wma_ref[...], preferred_element_type=jnp.float32)
    acc = (jnp.dot(g_ref[...], w1g_ref[...], preferred_element_type=jnp.float32)
           + b1_ref[...])
    su = su_ref[...]
    sm = sm_ref[...]
    for k in range(8):
        mu = (su == k).astype(jnp.float32)
        mm = (sm == k).astype(jnp.float32)
        acc += mu * pu[:, 128 * k:128 * (k + 1)]
        acc += mm * pm[:, 128 * k:128 * (k + 1)]
    h = jnp.maximum(acc, 0.0)
    o_ref[...] = jnp.sum(h * w2t_ref[...], axis=1) + b2_ref[...]


def _tc_mlp(wu, wm, genres, su, sm, wua, wma, w1g, b1, W2, b2):
    blk = 2048
    grid = (_B // blk,)
    full = lambda r, c: pl.BlockSpec((r, c), lambda i: (0, 0))
    return pl.pallas_call(
        _mlp_body,
        grid=grid,
        in_specs=[
            pl.BlockSpec((blk, 64), lambda i: (i, 0)),
            pl.BlockSpec((blk, 64), lambda i: (i, 0)),
            pl.BlockSpec((blk, 20), lambda i: (i, 0)),
            pl.BlockSpec((blk, 1), lambda i: (i, 0)),
            pl.BlockSpec((blk, 1), lambda i: (i, 0)),
            full(64, 1024),
            full(64, 1024),
            full(20, 128),
            full(1, 128),
            full(1, 128),
            pl.BlockSpec((1,), lambda i: (0,)),
        ],
        out_specs=pl.BlockSpec((blk,), lambda i: (i,)),
        out_shape=jax.ShapeDtypeStruct((_B,), jnp.float32),
    )(wu, wm, genres, su.reshape(_B, 1), sm.reshape(_B, 1),
      wua, wma, w1g, b1.reshape(1, 128), W2.reshape(1, 128), b2)


def _shifted_cat(w):
    return jnp.concatenate(
        [jnp.pad(w, ((2 * k, 14 - 2 * k), (0, 0))) for k in range(8)], axis=1)


def kernel(user, movie, genres, user_table, movie_table, W1, b1, W2, b2):
    ut_g = user_table.reshape(-1, _L)
    mt_g = movie_table.reshape(-1, _L)
    wu, wm, su, sm = _sc_gather()(user, movie, ut_g, mt_g)
    wua = _shifted_cat(W1[:_EMB])
    wma = _shifted_cat(W1[_EMB:2 * _EMB])
    return _tc_mlp(wu, wm, genres, su, sm, wua, wma, W1[2 * _EMB:],
                   b1, W2, b2)
